# Initial kernel scaffold; baseline (speedup 1.0000x reference)
import jax
import jax.numpy as jnp
from jax.experimental import pallas as pl


def kernel(inputs, targets):
    raise NotImplementedError("write your pallas kernel here")


# speedup vs baseline: 11.9049x; 11.9049x over previous
import functools

import jax
import jax.numpy as jnp
import numpy as np
from jax.experimental import pallas as pl
from jax.experimental.pallas import tpu as pltpu

M = 1024
N = 8192
K = N // 2 - 1
BLOCK_M = 128

_SIGN = np.int32(np.uint32(0x80000000))
_LOW31 = np.int32(0x7FFFFFFF)


def _softplus(x):
    return jnp.maximum(x, 0.0) + jnp.log1p(jnp.exp(-jnp.abs(x)))


def _mmcl_body(x_ref, tgt_ref, out_ref):
    pid = pl.program_id(0)
    x = x_ref[...]
    tgt = tgt_ref[pl.ds(pid * BLOCK_M, BLOCK_M), :]

    col = jax.lax.broadcasted_iota(jnp.int32, (BLOCK_M, N), 1)
    pos_mask = col == tgt
    neg_mask = jnp.logical_not(pos_mask)

    bits = jax.lax.bitcast_convert_type(x, jnp.int32)
    s = jnp.where(bits >= 0, bits, bits ^ _LOW31)

    p_u = jnp.zeros((BLOCK_M, 1), dtype=jnp.int32)
    for b in range(31, -1, -1):
        bit = np.int32(np.uint32(1 << b))
        c_u = p_u | bit
        c_s = c_u ^ _SIGN
        cnt = jnp.sum(
            jnp.where((s >= c_s) & neg_mask, 1, 0).astype(jnp.int32),
            axis=1,
            keepdims=True,
        )
        p_u = jnp.where(cnt >= K, c_u, p_u)

    t_s = p_u ^ _SIGN
    t_bits = jnp.where(t_s >= 0, t_s, t_s ^ _LOW31)
    t_f = jax.lax.bitcast_convert_type(t_bits, jnp.float32)

    gt = (s > t_s) & neg_mask
    c = jnp.sum(gt.astype(jnp.int32), axis=1, keepdims=True).astype(jnp.float32)
    sp = _softplus(x)
    sum_sp = jnp.sum(jnp.where(gt, sp, 0.0), axis=1, keepdims=True)
    l_neg = (sum_sp + (K - c) * _softplus(t_f)) * (1.0 / K)

    pos = jnp.sum(jnp.where(pos_mask, x, 0.0), axis=1, keepdims=True)
    per_row = _softplus(-pos) + l_neg

    @pl.when(pid == 0)
    def _():
        out_ref[...] = jnp.zeros((1, 1), jnp.float32)

    out_ref[...] += jnp.sum(per_row).reshape(1, 1) * (1.0 / M)


@jax.jit
def kernel(inputs, targets):
    tgt2d = targets.astype(jnp.int32).reshape(M, 1)
    grid = M // BLOCK_M
    out = pl.pallas_call(
        _mmcl_body,
        grid=(grid,),
        in_specs=[
            pl.BlockSpec((BLOCK_M, N), lambda i: (i, 0)),
            pl.BlockSpec((M, 1), lambda i: (0, 0)),
        ],
        out_specs=pl.BlockSpec((1, 1), lambda i: (0, 0)),
        out_shape=jax.ShapeDtypeStruct((1, 1), jnp.float32),
        compiler_params=pltpu.CompilerParams(
            dimension_semantics=("arbitrary",),
        ),
    )(inputs, tgt2d)
    return out[0, 0]
